# baseline (speedup 1.0000x reference)
import functools

import jax
import jax.numpy as jnp
from jax.experimental import pallas as pl

_N_RAYS = 4096
_N_SAMPLES = 192
_NEAR, _FAR = 2.0, 6.0
_STEP = (_FAR - _NEAR) / _N_SAMPLES
_LOG_EPS = -23.025850929940457
_BLOCK_R = 128


def _render_block(rays_ref, w1_ref, b1_ref, wcat_ref, bs_ref, br_ref,
                  rgb_ref, op_ref, depth_ref):
    f32 = jnp.float32
    rays = rays_ref[...]
    ro = rays[:, 0:3] * (2.0 / 3.0)
    rd = rays[:, 3:6] * (2.0 / 3.0)
    w1 = w1_ref[...]
    hi = jax.lax.Precision.HIGHEST
    a = jnp.dot(ro, w1, precision=hi, preferred_element_type=f32) + b1_ref[...]
    b = jnp.dot(rd, w1, precision=hi, preferred_element_type=f32)

    r = rays.shape[0]
    a3 = a[:, :, None]
    b3 = b[:, :, None]
    s_idx = jax.lax.broadcasted_iota(jnp.int32, (1, _N_SAMPLES), 1).astype(f32)
    t_mid2 = _NEAR + (s_idx + 0.5) * _STEP
    t_mid = t_mid2[:, None, :]
    h = jnp.maximum(a3 + t_mid * b3, 0.0).astype(jnp.bfloat16)

    wcat = jnp.broadcast_to(wcat_ref[...][None].astype(jnp.bfloat16), (r, 4, 128))
    z = jax.lax.dot_general(
        wcat, h, dimension_numbers=(((2,), (1,)), ((0,), (0,))),
        preferred_element_type=f32)

    sigma = jax.nn.softplus(z[:, 0, :] + bs_ref[0, 0])
    x = sigma * _STEP
    alpha = 1.0 - jnp.exp(-x)
    log_trans = jnp.maximum(-x, _LOG_EPS)
    rows = jax.lax.broadcasted_iota(jnp.int32, (_N_SAMPLES, _N_SAMPLES), 0)
    cols = jax.lax.broadcasted_iota(jnp.int32, (_N_SAMPLES, _N_SAMPLES), 1)
    tri = (rows < cols).astype(f32)
    excl = jnp.dot(log_trans, tri, precision=hi, preferred_element_type=f32)
    weights = alpha * jnp.exp(excl)

    outs = []
    for c in range(3):
        rgb_c = jax.nn.sigmoid(z[:, 1 + c, :] + br_ref[0, c])
        outs.append(jnp.sum(weights * rgb_c, axis=-1)[:, None])
    rgb_ref[...] = jnp.concatenate(outs, axis=1)
    op_ref[...] = jnp.sum(weights, axis=-1)[:, None]
    depth_ref[...] = jnp.sum(weights * t_mid2, axis=-1)[:, None]


@jax.jit
def kernel(rays, W1, b1, W_sigma, b_sigma, W_rgb, b_rgb):
    n_rays = rays.shape[0]
    wcat = jnp.concatenate([W_sigma, W_rgb], axis=1).T
    b1_2d = b1.reshape(1, -1)
    bs_2d = b_sigma.reshape(1, 1)
    br_2d = W_rgb.dtype.type(0) + b_rgb.reshape(1, 3)
    grid = (n_rays // _BLOCK_R,)
    rgb, op, depth = pl.pallas_call(
        _render_block,
        grid=grid,
        in_specs=[
            pl.BlockSpec((_BLOCK_R, 6), lambda i: (i, 0)),
            pl.BlockSpec((3, 128), lambda i: (0, 0)),
            pl.BlockSpec((1, 128), lambda i: (0, 0)),
            pl.BlockSpec((4, 128), lambda i: (0, 0)),
            pl.BlockSpec((1, 1), lambda i: (0, 0)),
            pl.BlockSpec((1, 3), lambda i: (0, 0)),
        ],
        out_specs=[
            pl.BlockSpec((_BLOCK_R, 3), lambda i: (i, 0)),
            pl.BlockSpec((_BLOCK_R, 1), lambda i: (i, 0)),
            pl.BlockSpec((_BLOCK_R, 1), lambda i: (i, 0)),
        ],
        out_shape=[
            jax.ShapeDtypeStruct((n_rays, 3), jnp.float32),
            jax.ShapeDtypeStruct((n_rays, 1), jnp.float32),
            jax.ShapeDtypeStruct((n_rays, 1), jnp.float32),
        ],
    )(rays, W1, b1_2d, wcat, bs_2d, br_2d)
    return rgb, op[:, 0], depth[:, 0]


# speedup vs baseline: 1.3199x; 1.3199x over previous
import jax
import jax.numpy as jnp
from jax.experimental import pallas as pl

_N_RAYS = 4096
_N_SAMPLES = 192
_NEAR, _FAR = 2.0, 6.0
_STEP = (_FAR - _NEAR) / _N_SAMPLES
_LOG_EPS = -23.025850929940457
_BLOCK_R = 128


def _render_block(rays_ref, w1_ref, b1_ref, wcat_ref, bs_ref, br_ref, tri_ref,
                  rgb_ref, op_ref, depth_ref):
    f32 = jnp.float32
    rays = rays_ref[...]
    ro = rays[:, 0:3] * (2.0 / 3.0)
    rd = rays[:, 3:6] * (2.0 / 3.0)
    w1 = w1_ref[...]
    hi = jax.lax.Precision.HIGHEST
    a = jnp.dot(ro, w1, precision=hi, preferred_element_type=f32) + b1_ref[...]
    b = jnp.dot(rd, w1, precision=hi, preferred_element_type=f32)

    r = rays.shape[0]
    c = jnp.concatenate([a[:, None, :], b[:, None, :]], axis=1)
    s_idx = jax.lax.broadcasted_iota(jnp.int32, (1, _N_SAMPLES), 1).astype(f32)
    t_mid2 = _NEAR + (s_idx + 0.5) * _STEP
    ones = jnp.ones((1, _N_SAMPLES), dtype=f32)
    tmat = jnp.concatenate([ones[:, None, :], t_mid2[:, None, :]], axis=1)
    tmat = jnp.broadcast_to(tmat, (r, 2, _N_SAMPLES))
    h = jax.lax.dot_general(
        c, tmat, dimension_numbers=(((1,), (1,)), ((0,), (0,))),
        preferred_element_type=f32)
    h = jnp.maximum(h, 0.0)

    wcat = jnp.broadcast_to(wcat_ref[...][None], (r, 4, 128))
    z = jax.lax.dot_general(
        wcat, h, dimension_numbers=(((2,), (1,)), ((0,), (0,))),
        preferred_element_type=f32)

    sigma = jax.nn.softplus(z[:, 0, :] + bs_ref[0, 0])
    x = sigma * _STEP
    alpha = 1.0 - jnp.exp(-x)
    log_trans = jnp.maximum(-x, _LOG_EPS)
    excl = jnp.dot(log_trans, tri_ref[...], precision=hi,
                   preferred_element_type=f32)
    weights = alpha * jnp.exp(excl)

    outs = []
    for ch in range(3):
        rgb_c = jax.nn.sigmoid(z[:, 1 + ch, :] + br_ref[0, ch])
        outs.append(jnp.sum(weights * rgb_c, axis=-1)[:, None])
    rgb_ref[...] = jnp.concatenate(outs, axis=1)
    op_ref[...] = jnp.sum(weights, axis=-1)[:, None]
    depth_ref[...] = jnp.sum(weights * t_mid2, axis=-1)[:, None]


@jax.jit
def kernel(rays, W1, b1, W_sigma, b_sigma, W_rgb, b_rgb):
    n_rays = rays.shape[0]
    wcat = jnp.concatenate([W_sigma, W_rgb], axis=1).T
    b1_2d = b1.reshape(1, -1)
    bs_2d = b_sigma.reshape(1, 1)
    br_2d = b_rgb.reshape(1, 3)
    s = _N_SAMPLES
    tri = (jnp.arange(s, dtype=jnp.int32)[:, None]
           < jnp.arange(s, dtype=jnp.int32)[None, :]).astype(jnp.float32)
    grid = (n_rays // _BLOCK_R,)
    rgb, op, depth = pl.pallas_call(
        _render_block,
        grid=grid,
        in_specs=[
            pl.BlockSpec((_BLOCK_R, 6), lambda i: (i, 0)),
            pl.BlockSpec((3, 128), lambda i: (0, 0)),
            pl.BlockSpec((1, 128), lambda i: (0, 0)),
            pl.BlockSpec((4, 128), lambda i: (0, 0)),
            pl.BlockSpec((1, 1), lambda i: (0, 0)),
            pl.BlockSpec((1, 3), lambda i: (0, 0)),
            pl.BlockSpec((s, s), lambda i: (0, 0)),
        ],
        out_specs=[
            pl.BlockSpec((_BLOCK_R, 3), lambda i: (i, 0)),
            pl.BlockSpec((_BLOCK_R, 1), lambda i: (i, 0)),
            pl.BlockSpec((_BLOCK_R, 1), lambda i: (i, 0)),
        ],
        out_shape=[
            jax.ShapeDtypeStruct((n_rays, 3), jnp.float32),
            jax.ShapeDtypeStruct((n_rays, 1), jnp.float32),
            jax.ShapeDtypeStruct((n_rays, 1), jnp.float32),
        ],
    )(rays, W1, b1_2d, wcat, bs_2d, br_2d, tri)
    return rgb, op[:, 0], depth[:, 0]


# speedup vs baseline: 1.4392x; 1.0904x over previous
import jax
import jax.numpy as jnp
from jax.experimental import pallas as pl

_N_RAYS = 4096
_N_SAMPLES = 192
_NEAR, _FAR = 2.0, 6.0
_STEP = (_FAR - _NEAR) / _N_SAMPLES
_LOG_EPS = -23.025850929940457
_BLOCK_R = 256


def _render_block(rays_ref, w1_ref, b1_ref, wcat_ref, bs_ref, br_ref, tri_ref,
                  rgb_ref, op_ref, depth_ref):
    f32 = jnp.float32
    rays = rays_ref[...]
    ro = rays[:, 0:3] * (2.0 / 3.0)
    rd = rays[:, 3:6] * (2.0 / 3.0)
    w1 = w1_ref[...]
    hi = jax.lax.Precision.HIGHEST
    a = jnp.dot(ro, w1, precision=hi, preferred_element_type=f32) + b1_ref[...]
    b = jnp.dot(rd, w1, precision=hi, preferred_element_type=f32)

    r = rays.shape[0]
    c = jnp.concatenate([a[:, None, :], b[:, None, :]], axis=1)
    s_idx = jax.lax.broadcasted_iota(jnp.int32, (1, _N_SAMPLES), 1).astype(f32)
    t_mid2 = _NEAR + (s_idx + 0.5) * _STEP
    ones = jnp.ones((1, _N_SAMPLES), dtype=f32)
    tmat = jnp.concatenate([ones[:, None, :], t_mid2[:, None, :]], axis=1)
    tmat = jnp.broadcast_to(tmat, (r, 2, _N_SAMPLES))
    h = jax.lax.dot_general(
        c, tmat, dimension_numbers=(((1,), (1,)), ((0,), (0,))),
        preferred_element_type=f32)
    h = jnp.maximum(h, 0.0)

    wcat = jnp.broadcast_to(wcat_ref[...][None], (r, 4, 128))
    z = jax.lax.dot_general(
        wcat, h, dimension_numbers=(((2,), (1,)), ((0,), (0,))),
        preferred_element_type=f32)

    sigma = jax.nn.softplus(z[:, 0, :] + bs_ref[0, 0])
    x = sigma * _STEP
    alpha = 1.0 - jnp.exp(-x)
    log_trans = jnp.maximum(-x, _LOG_EPS)
    excl = jnp.dot(log_trans, tri_ref[...], precision=hi,
                   preferred_element_type=f32)
    weights = alpha * jnp.exp(excl)

    outs = []
    for ch in range(3):
        rgb_c = jax.nn.sigmoid(z[:, 1 + ch, :] + br_ref[0, ch])
        outs.append(jnp.sum(weights * rgb_c, axis=-1)[:, None])
    rgb_ref[...] = jnp.concatenate(outs, axis=1)
    op_ref[...] = jnp.sum(weights, axis=-1)[:, None]
    depth_ref[...] = jnp.sum(weights * t_mid2, axis=-1)[:, None]


@jax.jit
def kernel(rays, W1, b1, W_sigma, b_sigma, W_rgb, b_rgb):
    n_rays = rays.shape[0]
    wcat = jnp.concatenate([W_sigma, W_rgb], axis=1).T
    b1_2d = b1.reshape(1, -1)
    bs_2d = b_sigma.reshape(1, 1)
    br_2d = b_rgb.reshape(1, 3)
    s = _N_SAMPLES
    tri = (jnp.arange(s, dtype=jnp.int32)[:, None]
           < jnp.arange(s, dtype=jnp.int32)[None, :]).astype(jnp.float32)
    grid = (n_rays // _BLOCK_R,)
    rgb, op, depth = pl.pallas_call(
        _render_block,
        grid=grid,
        in_specs=[
            pl.BlockSpec((_BLOCK_R, 6), lambda i: (i, 0)),
            pl.BlockSpec((3, 128), lambda i: (0, 0)),
            pl.BlockSpec((1, 128), lambda i: (0, 0)),
            pl.BlockSpec((4, 128), lambda i: (0, 0)),
            pl.BlockSpec((1, 1), lambda i: (0, 0)),
            pl.BlockSpec((1, 3), lambda i: (0, 0)),
            pl.BlockSpec((s, s), lambda i: (0, 0)),
        ],
        out_specs=[
            pl.BlockSpec((_BLOCK_R, 3), lambda i: (i, 0)),
            pl.BlockSpec((_BLOCK_R, 1), lambda i: (i, 0)),
            pl.BlockSpec((_BLOCK_R, 1), lambda i: (i, 0)),
        ],
        out_shape=[
            jax.ShapeDtypeStruct((n_rays, 3), jnp.float32),
            jax.ShapeDtypeStruct((n_rays, 1), jnp.float32),
            jax.ShapeDtypeStruct((n_rays, 1), jnp.float32),
        ],
    )(rays, W1, b1_2d, wcat, bs_2d, br_2d, tri)
    return rgb, op[:, 0], depth[:, 0]
